# baseline (speedup 1.0000x reference)
import functools

import jax
import jax.numpy as jnp
from jax import lax
from jax.experimental import pallas as pl
from jax.experimental.pallas import tpu as pltpu
from jax.experimental.pallas import tpu_sc as plsc

R = 32
V = R * R * R


def _coords_body(pts_ref, vc_ref, idx_ref):
    p = pts_ref[0]
    mean = jnp.mean(p, axis=1, keepdims=True)
    nc = p - mean
    nrm = jnp.sqrt(nc[0:1] ** 2 + nc[1:2] ** 2 + nc[2:3] ** 2)
    denom = jnp.max(nrm) * 2.0
    vc = jnp.clip((nc / denom + 0.5) * R, 0.0, float(R - 1))
    vc_ref[0] = vc
    ri = jnp.round(vc).astype(jnp.int32)
    idx_ref[0] = ri[0:1] * (R * R) + ri[1:2] * R + ri[2:3]


def _compute_coords(pts_t):
    B, _, N = pts_t.shape
    return pl.pallas_call(
        _coords_body,
        grid=(B,),
        in_specs=[pl.BlockSpec((1, 3, N), lambda b: (b, 0, 0))],
        out_specs=[
            pl.BlockSpec((1, 3, N), lambda b: (b, 0, 0)),
            pl.BlockSpec((1, 1, N), lambda b: (b, 0, 0)),
        ],
        out_shape=[
            jax.ShapeDtypeStruct((B, 3, N), jnp.float32),
            jax.ShapeDtypeStruct((B, 1, N), jnp.int32),
        ],
    )(pts_t)


_CHUNK = 16384


def _sc_scatter_body(feat_hbm, idx_hbm, out_hbm, idx_v, feat_v, acc_v, inv_v,
                     *, B, C, N, groups):
    cpg = C // groups
    wid = lax.axis_index("s") * 2 + lax.axis_index("c")
    b = wid // groups
    g = wid % groups

    pltpu.sync_copy(idx_hbm.at[b], idx_v)

    zeros16 = jnp.zeros((16,), jnp.float32)
    ones16 = jnp.ones((16,), jnp.float32)

    def zero_body(i, _):
        acc_v[pl.ds(i * 16, 16)] = zeros16
        return 0

    lax.fori_loop(0, V // 16, zero_body, 0)

    def cnt_body(i, _):
        ix = idx_v[pl.ds(i * 16, 16)]
        plsc.addupdate_scatter(acc_v, [ix], ones16)
        return 0

    lax.fori_loop(0, N // 16, cnt_body, 0)

    def inv_body(i, _):
        c16 = acc_v[pl.ds(i * 16, 16)]
        inv_v[pl.ds(i * 16, 16)] = 1.0 / jnp.maximum(c16, 1.0)
        acc_v[pl.ds(i * 16, 16)] = zeros16
        return 0

    lax.fori_loop(0, V // 16, inv_body, 0)

    def chan_body(c, _):
        row = b * C + g * cpg + c

        def chunk_body(k, _):
            pltpu.sync_copy(feat_hbm.at[row, pl.ds(k * _CHUNK, _CHUNK)], feat_v)

            def sc_body(i, _):
                f = feat_v[pl.ds(i * 16, 16)]
                ix = idx_v[pl.ds(k * _CHUNK + i * 16, 16)]
                plsc.addupdate_scatter(acc_v, [ix], f)
                return 0

            lax.fori_loop(0, _CHUNK // 16, sc_body, 0)
            return 0

        lax.fori_loop(0, N // _CHUNK, chunk_body, 0)

        def mul_body(i, _):
            acc_v[pl.ds(i * 16, 16)] = acc_v[pl.ds(i * 16, 16)] * inv_v[pl.ds(i * 16, 16)]
            return 0

        lax.fori_loop(0, V // 16, mul_body, 0)
        pltpu.sync_copy(acc_v, out_hbm.at[row])
        lax.fori_loop(0, V // 16, zero_body, 0)
        return 0

    lax.fori_loop(0, cpg, chan_body, 0)


def _sc_scatter(feat2d, idx):
    BC, N = feat2d.shape
    B = idx.shape[0]
    C = BC // B
    groups = 32 // B
    body = functools.partial(_sc_scatter_body, B=B, C=C, N=N, groups=groups)
    mesh = plsc.VectorSubcoreMesh(core_axis_name="c", subcore_axis_name="s")
    return pl.kernel(
        body,
        out_type=jax.ShapeDtypeStruct((BC, V), jnp.float32),
        mesh=mesh,
        compiler_params=pltpu.CompilerParams(needs_layout_passes=False),
        scratch_types=[
            pltpu.VMEM((N,), jnp.int32),
            pltpu.VMEM((_CHUNK,), jnp.float32),
            pltpu.VMEM((V,), jnp.float32),
            pltpu.VMEM((V,), jnp.float32),
        ],
    )(feat2d, idx)


def kernel(features, points):
    B, C, N = features.shape
    pts_t = jnp.transpose(points, (0, 2, 1))
    vc, idx3 = _compute_coords(pts_t)
    out2d = _sc_scatter(features.reshape(B * C, N), idx3.reshape(B, N))
    voxel_feats = out2d.reshape(B, C, R, R, R)
    return voxel_feats, vc
